# baseline (speedup 1.0000x reference)
import functools

import jax
import jax.numpy as jnp
from jax import lax
from jax.experimental import pallas as pl
from jax.experimental.pallas import tpu as pltpu
from jax.experimental.pallas import tpu_sc as plsc

N = 10000
E = 320000
H = 128
NC = 2
NS = 16
NW = NC * NS
K = 128
EPT = 10240
BATCHES = EPT // K
B0 = 60
B1 = 100
E_PAD = NW * EPT
GARBAGE_ROW = N
AI_ROWS = N + 16
RPT = 624

PERM = tuple(
    [32 * (c // 16) + (c % 16) for c in range(H // 2)]
    + [32 * (c // 16) + 16 + (c % 16) for c in range(H // 2)]
)


def _mm_body(x_ref, wt_ref, b_ref, wtp_ref, bp_ref, o_ref, ob_ref):
    xb = x_ref[...]
    acc = jnp.dot(xb, wt_ref[...], preferred_element_type=jnp.float32)
    o_ref[...] = jnp.maximum(acc + b_ref[...], 0.0)
    accp = jnp.dot(xb, wtp_ref[...], preferred_element_type=jnp.float32)
    accp = jnp.maximum(accp + bp_ref[...], 0.0)
    bits = lax.bitcast_convert_type(accp, jnp.int32)
    rnd = bits + 0x7FFF + (lax.shift_right_logical(bits, 16) & 1)
    bf = lax.shift_right_logical(rnd, 16)
    packed = bf[:, :H // 2] | lax.shift_left(bf[:, H // 2:], 16)
    ob_ref[...] = lax.bitcast_convert_type(packed, jnp.float32)


def _matmul_relu(x, wt, b2, wtp, bp):
    return pl.pallas_call(
        _mm_body,
        grid=(20,),
        in_specs=[
            pl.BlockSpec((1000, H), lambda i: (i, 0)),
            pl.BlockSpec((H, H), lambda i: (0, 0)),
            pl.BlockSpec((1, H), lambda i: (0, 0)),
            pl.BlockSpec((H, H), lambda i: (0, 0)),
            pl.BlockSpec((1, H), lambda i: (0, 0)),
        ],
        out_specs=[
            pl.BlockSpec((1000, H), lambda i: (i, 0)),
            pl.BlockSpec((1000, H // 2), lambda i: (i, 0)),
        ],
        out_shape=[
            jax.ShapeDtypeStruct((2 * N, H), jnp.float32),
            jax.ShapeDtypeStruct((2 * N, H // 2), jnp.float32),
        ],
    )(x, wt, b2, wtp, bp)


def _sc_body(idx_hbm, table_hbm, out_hbm,
             crb0, crb1, gp0, gp1, fbuf, ai_sh, semg0, semg1):
    cid = lax.axis_index("c")
    sid = lax.axis_index("s")
    wid = cid * NS + sid

    zero16 = jnp.zeros((16,), jnp.float32)

    def _zrow(r, c):
        for j in range(H // 16):
            fbuf[r, pl.ds(j * 16, 16)] = zero16
        return c

    lax.fori_loop(0, K, _zrow, 0)
    for k in range(RPT // K):
        pltpu.sync_copy(fbuf, ai_sh.at[pl.ds(sid * RPT + k * K, K)])
    _rem = RPT - (RPT // K) * K
    pltpu.sync_copy(fbuf.at[pl.ds(0, _rem)],
                    ai_sh.at[pl.ds(sid * RPT + (RPT // K) * K, _rem)])

    @pl.when(sid == NS - 1)
    def _zero_tail():
        pltpu.sync_copy(fbuf.at[pl.ds(0, AI_ROWS - NS * RPT)],
                        ai_sh.at[pl.ds(NS * RPT, AI_ROWS - NS * RPT)])

    plsc.subcore_barrier()

    def _gather(g, sem, crb):
        pltpu.async_copy(table_hbm.at[crb.at[0]], g, sem)

    def _wait_g(g, sem):
        pltpu.make_async_copy(table_hbm.at[crb0.at[0]], g, sem).wait()

    himask = jnp.full((16,), -65536, jnp.int32)

    def _expand(g):
        @plsc.parallel_loop(0, K, 8)
        def _rows(t):
            base = pl.multiple_of(t, 8)
            for k in range(8):
                r = base + k
                for grp in range(H // 32):
                    v = plsc.bitcast(g[r, pl.ds(16 * grp, 16)], jnp.int32)
                    fbuf[r, pl.ds(32 * grp, 16)] = plsc.bitcast(
                        lax.shift_left(v, 16), jnp.float32)
                    fbuf[r, pl.ds(32 * grp + 16, 16)] = plsc.bitcast(
                        v & himask, jnp.float32)

    def _scatter(crb):
        pltpu.sync_copy(fbuf, ai_sh.at[crb.at[1]], add=True)

    nb = jnp.where(cid == 0, B0, B1)
    tbase = jnp.where(cid == 0, sid * B0, NS * B0 + sid * B1)

    def _load_idx(jt, crb):
        pltpu.sync_copy(idx_hbm.at[tbase + jt], crb)

    _load_idx(0, crb0)
    _gather(gp0, semg0, crb0)
    _load_idx(1, crb1)
    _gather(gp1, semg1, crb1)

    def _edge_body(t, c):
        j = 2 * t
        _wait_g(gp0, semg0)
        _expand(gp0)
        _scatter(crb0)
        _load_idx(j + 2, crb0)
        _gather(gp0, semg0, crb0)

        _wait_g(gp1, semg1)
        _expand(gp1)
        _scatter(crb1)
        _load_idx(j + 3, crb1)
        _gather(gp1, semg1, crb1)
        return c

    lax.fori_loop(0, nb // 2 - 1, _edge_body, 0)
    _wait_g(gp0, semg0)
    _expand(gp0)
    _scatter(crb0)
    _wait_g(gp1, semg1)
    _expand(gp1)
    _scatter(crb1)

    plsc.subcore_barrier()
    pltpu.sync_copy(ai_sh.at[pl.ds(sid * RPT, RPT)],
                    out_hbm.at[pl.ds(cid * N + sid * RPT, RPT)])

    @pl.when(sid == NS - 1)
    def _write_tail():
        pltpu.sync_copy(ai_sh.at[pl.ds(NS * RPT, N - NS * RPT)],
                        out_hbm.at[pl.ds(cid * N + NS * RPT, N - NS * RPT)])


@functools.cache
def _sc_scatter():
    return pl.kernel(
        _sc_body,
        out_type=jax.ShapeDtypeStruct((NC * N, H), jnp.float32),
        mesh=plsc.VectorSubcoreMesh(core_axis_name="c", subcore_axis_name="s"),
        compiler_params=pltpu.CompilerParams(
            needs_layout_passes=False, use_tc_tiling_on_sc=False),
        scratch_types=[
            pltpu.VMEM((2, K), jnp.int32),
            pltpu.VMEM((2, K), jnp.int32),
            pltpu.VMEM((K, H // 2), jnp.float32),
            pltpu.VMEM((K, H // 2), jnp.float32),
            pltpu.VMEM((K, H), jnp.float32),
            pltpu.VMEM_SHARED((AI_ROWS, H), jnp.float32),
            pltpu.SemaphoreType.DMA,
            pltpu.SemaphoreType.DMA,
        ],
    )


def _fin_body(s_ref, i_ref, ai0_ref, ai1_ref, x4_ref, lnw_ref, lnb_ref,
              o_ref):
    s = s_ref[...]
    i = i_ref[...]
    ai = ai0_ref[...] + ai1_ref[...]
    x4 = x4_ref[...]
    beta = x4[:, 0:1]
    gamma = x4[:, 1:2]
    ds = -beta * (ai * s)
    di = -ds - gamma * i
    dr = gamma * i
    w = lnw_ref[...]
    b = lnb_ref[...]

    def _ln(v):
        m = jnp.mean(v, axis=-1, keepdims=True)
        cvar = v - m
        var = jnp.mean(cvar * cvar, axis=-1, keepdims=True)
        return cvar * lax.rsqrt(var + 1e-5) * w + b

    o_ref[0] = _ln(ds)
    o_ref[1] = _ln(di)
    o_ref[2] = _ln(dr)
    o_ref[3] = x4


def _finalize(sir, ai_partials, x, lnw2, lnb2):
    return pl.pallas_call(
        _fin_body,
        grid=(10,),
        in_specs=[
            pl.BlockSpec((1000, H), lambda j: (j, 0)),
            pl.BlockSpec((1000, H), lambda j: (10 + j, 0)),
            pl.BlockSpec((1000, H), lambda j: (j, 0)),
            pl.BlockSpec((1000, H), lambda j: (10 + j, 0)),
            pl.BlockSpec((1000, H), lambda j: (30 + j, 0)),
            pl.BlockSpec((1, H), lambda j: (0, 0)),
            pl.BlockSpec((1, H), lambda j: (0, 0)),
        ],
        out_specs=pl.BlockSpec((4, 1000, H), lambda j: (0, j, 0)),
        out_shape=jax.ShapeDtypeStruct((4, N, H), jnp.float32),
    )(sir, sir, ai_partials, ai_partials, x, lnw2, lnb2)


def kernel(t, x, edge_index, W, b, ln_w, ln_b):
    del t
    wt = W.T
    perm = jnp.asarray(PERM, jnp.int32)
    sir, tbl = _matmul_relu(x, wt, b.reshape(1, H),
                            wt[:, perm], b[perm].reshape(1, H))

    n_pad = E_PAD - E
    rows = jnp.concatenate(
        [edge_index[0], jnp.full((n_pad,), GARBAGE_ROW, jnp.int32)])
    cols = jnp.concatenate(
        [edge_index[1] + jnp.int32(N), jnp.zeros((n_pad,), jnp.int32)])
    idx = jnp.stack([cols.reshape(E_PAD // K, K),
                     rows.reshape(E_PAD // K, K)], axis=1)

    ai_partials = _sc_scatter()(idx, tbl)

    out = _finalize(sir, ai_partials, x,
                    ln_w.reshape(1, H), ln_b.reshape(1, H))
    return out.reshape(4 * N, H)


# speedup vs baseline: 1.1961x; 1.1961x over previous
import functools

import jax
import jax.numpy as jnp
from jax import lax
from jax.experimental import pallas as pl
from jax.experimental.pallas import tpu as pltpu
from jax.experimental.pallas import tpu_sc as plsc

N = 10000
E = 320000
H = 128
NC = 2
NS = 16
NW = NC * NS
K = 128
EPT = 10240
BATCHES = EPT // K
B0 = 100
B1 = 60
E_PAD = NW * EPT
GARBAGE_ROW = N
AI_ROWS = N + 16
RPT = 624

PERM = tuple(
    [32 * (c // 16) + (c % 16) for c in range(H // 2)]
    + [32 * (c // 16) + 16 + (c % 16) for c in range(H // 2)]
)


def _mm_body(x_ref, wt_ref, b_ref, wtp_ref, bp_ref, o_ref, ob_ref):
    xb = x_ref[...]
    acc = jnp.dot(xb, wt_ref[...], preferred_element_type=jnp.float32)
    o_ref[...] = jnp.maximum(acc + b_ref[...], 0.0)
    accp = jnp.dot(xb, wtp_ref[...], preferred_element_type=jnp.float32)
    accp = jnp.maximum(accp + bp_ref[...], 0.0)
    bits = lax.bitcast_convert_type(accp, jnp.int32)
    rnd = bits + 0x7FFF + (lax.shift_right_logical(bits, 16) & 1)
    bf = lax.shift_right_logical(rnd, 16)
    packed = bf[:, :H // 2] | lax.shift_left(bf[:, H // 2:], 16)
    ob_ref[...] = lax.bitcast_convert_type(packed, jnp.float32)


def _matmul_relu(x, wt, b2, wtp, bp):
    return pl.pallas_call(
        _mm_body,
        grid=(20,),
        in_specs=[
            pl.BlockSpec((1000, H), lambda i: (i, 0)),
            pl.BlockSpec((H, H), lambda i: (0, 0)),
            pl.BlockSpec((1, H), lambda i: (0, 0)),
            pl.BlockSpec((H, H), lambda i: (0, 0)),
            pl.BlockSpec((1, H), lambda i: (0, 0)),
        ],
        out_specs=[
            pl.BlockSpec((1000, H), lambda i: (i, 0)),
            pl.BlockSpec((1000, H // 2), lambda i: (i, 0)),
        ],
        out_shape=[
            jax.ShapeDtypeStruct((2 * N, H), jnp.float32),
            jax.ShapeDtypeStruct((2 * N, H // 2), jnp.float32),
        ],
    )(x, wt, b2, wtp, bp)


def _sc_body(idx_hbm, table_hbm, out_hbm,
             crb0, crb1, gp0, gp1, fbuf, ai_sh, semg0, semg1):
    cid = lax.axis_index("c")
    sid = lax.axis_index("s")
    wid = cid * NS + sid

    zero16 = jnp.zeros((16,), jnp.float32)

    def _zrow(r, c):
        for j in range(H // 16):
            fbuf[r, pl.ds(j * 16, 16)] = zero16
        return c

    lax.fori_loop(0, K, _zrow, 0)
    for k in range(RPT // K):
        pltpu.sync_copy(fbuf, ai_sh.at[pl.ds(sid * RPT + k * K, K)])
    _rem = RPT - (RPT // K) * K
    pltpu.sync_copy(fbuf.at[pl.ds(0, _rem)],
                    ai_sh.at[pl.ds(sid * RPT + (RPT // K) * K, _rem)])

    @pl.when(sid == NS - 1)
    def _zero_tail():
        pltpu.sync_copy(fbuf.at[pl.ds(0, AI_ROWS - NS * RPT)],
                        ai_sh.at[pl.ds(NS * RPT, AI_ROWS - NS * RPT)])

    plsc.subcore_barrier()

    def _gather(g, sem, crb):
        pltpu.async_copy(table_hbm.at[crb.at[0]], g, sem)

    def _wait_g(g, sem):
        pltpu.make_async_copy(table_hbm.at[crb0.at[0]], g, sem).wait()

    himask = jnp.full((16,), -65536, jnp.int32)

    def _expand(g):
        @plsc.parallel_loop(0, K, 8)
        def _rows(t):
            base = pl.multiple_of(t, 8)
            for k in range(8):
                r = base + k
                for grp in range(H // 32):
                    v = plsc.bitcast(g[r, pl.ds(16 * grp, 16)], jnp.int32)
                    fbuf[r, pl.ds(32 * grp, 16)] = plsc.bitcast(
                        lax.shift_left(v, 16), jnp.float32)
                    fbuf[r, pl.ds(32 * grp + 16, 16)] = plsc.bitcast(
                        v & himask, jnp.float32)

    def _scatter(crb):
        pltpu.sync_copy(fbuf, ai_sh.at[crb.at[1]], add=True)

    nb = jnp.where(cid == 0, B0, B1)
    tbase = jnp.where(cid == 0, sid * B0, NS * B0 + sid * B1)

    def _load_idx(jt, crb):
        pltpu.sync_copy(idx_hbm.at[tbase + jt], crb)

    _load_idx(0, crb0)
    _gather(gp0, semg0, crb0)
    _load_idx(1, crb1)
    _gather(gp1, semg1, crb1)

    def _edge_body(t, c):
        j = 2 * t
        _wait_g(gp0, semg0)
        _expand(gp0)
        _scatter(crb0)
        _load_idx(j + 2, crb0)
        _gather(gp0, semg0, crb0)

        _wait_g(gp1, semg1)
        _expand(gp1)
        _scatter(crb1)
        _load_idx(j + 3, crb1)
        _gather(gp1, semg1, crb1)
        return c

    lax.fori_loop(0, nb // 2 - 1, _edge_body, 0)
    _wait_g(gp0, semg0)
    _expand(gp0)
    _scatter(crb0)
    _wait_g(gp1, semg1)
    _expand(gp1)
    _scatter(crb1)

    plsc.subcore_barrier()
    pltpu.sync_copy(ai_sh.at[pl.ds(sid * RPT, RPT)],
                    out_hbm.at[pl.ds(cid * N + sid * RPT, RPT)])

    @pl.when(sid == NS - 1)
    def _write_tail():
        pltpu.sync_copy(ai_sh.at[pl.ds(NS * RPT, N - NS * RPT)],
                        out_hbm.at[pl.ds(cid * N + NS * RPT, N - NS * RPT)])


@functools.cache
def _sc_scatter():
    return pl.kernel(
        _sc_body,
        out_type=jax.ShapeDtypeStruct((NC * N, H), jnp.float32),
        mesh=plsc.VectorSubcoreMesh(core_axis_name="c", subcore_axis_name="s"),
        compiler_params=pltpu.CompilerParams(
            needs_layout_passes=False, use_tc_tiling_on_sc=False),
        scratch_types=[
            pltpu.VMEM((2, K), jnp.int32),
            pltpu.VMEM((2, K), jnp.int32),
            pltpu.VMEM((K, H // 2), jnp.float32),
            pltpu.VMEM((K, H // 2), jnp.float32),
            pltpu.VMEM((K, H), jnp.float32),
            pltpu.VMEM_SHARED((AI_ROWS, H), jnp.float32),
            pltpu.SemaphoreType.DMA,
            pltpu.SemaphoreType.DMA,
        ],
    )


def _fin_body(s_ref, i_ref, ai0_ref, ai1_ref, x4_ref, lnw_ref, lnb_ref,
              o_ref):
    s = s_ref[...]
    i = i_ref[...]
    ai = ai0_ref[...] + ai1_ref[...]
    x4 = x4_ref[...]
    beta = x4[:, 0:1]
    gamma = x4[:, 1:2]
    ds = -beta * (ai * s)
    di = -ds - gamma * i
    dr = gamma * i
    w = lnw_ref[...]
    b = lnb_ref[...]

    def _ln(v):
        m = jnp.mean(v, axis=-1, keepdims=True)
        cvar = v - m
        var = jnp.mean(cvar * cvar, axis=-1, keepdims=True)
        return cvar * lax.rsqrt(var + 1e-5) * w + b

    o_ref[0] = _ln(ds)
    o_ref[1] = _ln(di)
    o_ref[2] = _ln(dr)
    o_ref[3] = x4


def _finalize(sir, ai_partials, x, lnw2, lnb2):
    return pl.pallas_call(
        _fin_body,
        grid=(10,),
        in_specs=[
            pl.BlockSpec((1000, H), lambda j: (j, 0)),
            pl.BlockSpec((1000, H), lambda j: (10 + j, 0)),
            pl.BlockSpec((1000, H), lambda j: (j, 0)),
            pl.BlockSpec((1000, H), lambda j: (10 + j, 0)),
            pl.BlockSpec((1000, H), lambda j: (30 + j, 0)),
            pl.BlockSpec((1, H), lambda j: (0, 0)),
            pl.BlockSpec((1, H), lambda j: (0, 0)),
        ],
        out_specs=pl.BlockSpec((4, 1000, H), lambda j: (0, j, 0)),
        out_shape=jax.ShapeDtypeStruct((4, N, H), jnp.float32),
    )(sir, sir, ai_partials, ai_partials, x, lnw2, lnb2)


def kernel(t, x, edge_index, W, b, ln_w, ln_b):
    del t
    wt = W.T
    perm = jnp.asarray(PERM, jnp.int32)
    sir, tbl = _matmul_relu(x, wt, b.reshape(1, H),
                            wt[:, perm], b[perm].reshape(1, H))

    n_pad = E_PAD - E
    rows = jnp.concatenate(
        [edge_index[0], jnp.full((n_pad,), GARBAGE_ROW, jnp.int32)])
    cols = jnp.concatenate(
        [edge_index[1] + jnp.int32(N), jnp.zeros((n_pad,), jnp.int32)])
    idx = jnp.stack([cols.reshape(E_PAD // K, K),
                     rows.reshape(E_PAD // K, K)], axis=1)

    ai_partials = _sc_scatter()(idx, tbl)

    out = _finalize(sir, ai_partials, x,
                    ln_w.reshape(1, H), ln_b.reshape(1, H))
    return out.reshape(4 * N, H)


# speedup vs baseline: 1.2000x; 1.0033x over previous
import functools

import jax
import jax.numpy as jnp
from jax import lax
from jax.experimental import pallas as pl
from jax.experimental.pallas import tpu as pltpu
from jax.experimental.pallas import tpu_sc as plsc

N = 10000
E = 320000
H = 128
NC = 2
NS = 16
NW = NC * NS
K = 128
EPT = 10240
BATCHES = EPT // K
B0 = 104
B1 = 56
E_PAD = NW * EPT
GARBAGE_ROW = N
AI_ROWS = N + 16
RPT = 624

PERM = tuple(
    [32 * (c // 16) + (c % 16) for c in range(H // 2)]
    + [32 * (c // 16) + 16 + (c % 16) for c in range(H // 2)]
)


def _mm_body(x_ref, wt_ref, b_ref, wtp_ref, bp_ref, o_ref, ob_ref):
    xb = x_ref[...]
    acc = jnp.dot(xb, wt_ref[...], preferred_element_type=jnp.float32)
    o_ref[...] = jnp.maximum(acc + b_ref[...], 0.0)
    accp = jnp.dot(xb, wtp_ref[...], preferred_element_type=jnp.float32)
    accp = jnp.maximum(accp + bp_ref[...], 0.0)
    bits = lax.bitcast_convert_type(accp, jnp.int32)
    rnd = bits + 0x7FFF + (lax.shift_right_logical(bits, 16) & 1)
    bf = lax.shift_right_logical(rnd, 16)
    packed = bf[:, :H // 2] | lax.shift_left(bf[:, H // 2:], 16)
    ob_ref[...] = lax.bitcast_convert_type(packed, jnp.float32)


def _matmul_relu(x, wt, b2, wtp, bp):
    return pl.pallas_call(
        _mm_body,
        grid=(20,),
        in_specs=[
            pl.BlockSpec((1000, H), lambda i: (i, 0)),
            pl.BlockSpec((H, H), lambda i: (0, 0)),
            pl.BlockSpec((1, H), lambda i: (0, 0)),
            pl.BlockSpec((H, H), lambda i: (0, 0)),
            pl.BlockSpec((1, H), lambda i: (0, 0)),
        ],
        out_specs=[
            pl.BlockSpec((1000, H), lambda i: (i, 0)),
            pl.BlockSpec((1000, H // 2), lambda i: (i, 0)),
        ],
        out_shape=[
            jax.ShapeDtypeStruct((2 * N, H), jnp.float32),
            jax.ShapeDtypeStruct((2 * N, H // 2), jnp.float32),
        ],
    )(x, wt, b2, wtp, bp)


def _sc_body(idx_hbm, table_hbm, out_hbm,
             crb0, crb1, gp0, gp1, fbuf, ai_sh, semg0, semg1):
    cid = lax.axis_index("c")
    sid = lax.axis_index("s")
    wid = cid * NS + sid

    zero16 = jnp.zeros((16,), jnp.float32)

    def _zrow(r, c):
        for j in range(H // 16):
            fbuf[r, pl.ds(j * 16, 16)] = zero16
        return c

    lax.fori_loop(0, K, _zrow, 0)
    for k in range(RPT // K):
        pltpu.sync_copy(fbuf, ai_sh.at[pl.ds(sid * RPT + k * K, K)])
    _rem = RPT - (RPT // K) * K
    pltpu.sync_copy(fbuf.at[pl.ds(0, _rem)],
                    ai_sh.at[pl.ds(sid * RPT + (RPT // K) * K, _rem)])

    @pl.when(sid == NS - 1)
    def _zero_tail():
        pltpu.sync_copy(fbuf.at[pl.ds(0, AI_ROWS - NS * RPT)],
                        ai_sh.at[pl.ds(NS * RPT, AI_ROWS - NS * RPT)])

    plsc.subcore_barrier()

    def _gather(g, sem, crb):
        pltpu.async_copy(table_hbm.at[crb.at[0]], g, sem)

    def _wait_g(g, sem):
        pltpu.make_async_copy(table_hbm.at[crb0.at[0]], g, sem).wait()

    himask = jnp.full((16,), -65536, jnp.int32)

    def _expand(g):
        @plsc.parallel_loop(0, K, 8)
        def _rows(t):
            base = pl.multiple_of(t, 8)
            for k in range(8):
                r = base + k
                for grp in range(H // 32):
                    v = plsc.bitcast(g[r, pl.ds(16 * grp, 16)], jnp.int32)
                    fbuf[r, pl.ds(32 * grp, 16)] = plsc.bitcast(
                        lax.shift_left(v, 16), jnp.float32)
                    fbuf[r, pl.ds(32 * grp + 16, 16)] = plsc.bitcast(
                        v & himask, jnp.float32)

    def _scatter(crb):
        pltpu.sync_copy(fbuf, ai_sh.at[crb.at[1]], add=True)

    nb = jnp.where(cid == 0, B0, B1)
    tbase = jnp.where(cid == 0, sid * B0, NS * B0 + sid * B1)

    def _load_idx(jt, crb):
        pltpu.sync_copy(idx_hbm.at[tbase + jt], crb)

    _load_idx(0, crb0)
    _gather(gp0, semg0, crb0)
    _load_idx(1, crb1)
    _gather(gp1, semg1, crb1)

    def _edge_body(t, c):
        j = 2 * t
        _wait_g(gp0, semg0)
        _expand(gp0)
        _scatter(crb0)
        _load_idx(j + 2, crb0)
        _gather(gp0, semg0, crb0)

        _wait_g(gp1, semg1)
        _expand(gp1)
        _scatter(crb1)
        _load_idx(j + 3, crb1)
        _gather(gp1, semg1, crb1)
        return c

    lax.fori_loop(0, nb // 2 - 1, _edge_body, 0)
    _wait_g(gp0, semg0)
    _expand(gp0)
    _scatter(crb0)
    _wait_g(gp1, semg1)
    _expand(gp1)
    _scatter(crb1)

    plsc.subcore_barrier()
    pltpu.sync_copy(ai_sh.at[pl.ds(sid * RPT, RPT)],
                    out_hbm.at[pl.ds(cid * N + sid * RPT, RPT)])

    @pl.when(sid == NS - 1)
    def _write_tail():
        pltpu.sync_copy(ai_sh.at[pl.ds(NS * RPT, N - NS * RPT)],
                        out_hbm.at[pl.ds(cid * N + NS * RPT, N - NS * RPT)])


@functools.cache
def _sc_scatter():
    return pl.kernel(
        _sc_body,
        out_type=jax.ShapeDtypeStruct((NC * N, H), jnp.float32),
        mesh=plsc.VectorSubcoreMesh(core_axis_name="c", subcore_axis_name="s"),
        compiler_params=pltpu.CompilerParams(
            needs_layout_passes=False, use_tc_tiling_on_sc=False),
        scratch_types=[
            pltpu.VMEM((2, K), jnp.int32),
            pltpu.VMEM((2, K), jnp.int32),
            pltpu.VMEM((K, H // 2), jnp.float32),
            pltpu.VMEM((K, H // 2), jnp.float32),
            pltpu.VMEM((K, H), jnp.float32),
            pltpu.VMEM_SHARED((AI_ROWS, H), jnp.float32),
            pltpu.SemaphoreType.DMA,
            pltpu.SemaphoreType.DMA,
        ],
    )


def _fin_body(s_ref, i_ref, ai0_ref, ai1_ref, x4_ref, lnw_ref, lnb_ref,
              o_ref):
    s = s_ref[...]
    i = i_ref[...]
    ai = ai0_ref[...] + ai1_ref[...]
    x4 = x4_ref[...]
    beta = x4[:, 0:1]
    gamma = x4[:, 1:2]
    ds = -beta * (ai * s)
    di = -ds - gamma * i
    dr = gamma * i
    w = lnw_ref[...]
    b = lnb_ref[...]

    def _ln(v):
        m = jnp.mean(v, axis=-1, keepdims=True)
        cvar = v - m
        var = jnp.mean(cvar * cvar, axis=-1, keepdims=True)
        return cvar * lax.rsqrt(var + 1e-5) * w + b

    o_ref[0] = _ln(ds)
    o_ref[1] = _ln(di)
    o_ref[2] = _ln(dr)
    o_ref[3] = x4


def _finalize(sir, ai_partials, x, lnw2, lnb2):
    return pl.pallas_call(
        _fin_body,
        grid=(10,),
        in_specs=[
            pl.BlockSpec((1000, H), lambda j: (j, 0)),
            pl.BlockSpec((1000, H), lambda j: (10 + j, 0)),
            pl.BlockSpec((1000, H), lambda j: (j, 0)),
            pl.BlockSpec((1000, H), lambda j: (10 + j, 0)),
            pl.BlockSpec((1000, H), lambda j: (30 + j, 0)),
            pl.BlockSpec((1, H), lambda j: (0, 0)),
            pl.BlockSpec((1, H), lambda j: (0, 0)),
        ],
        out_specs=pl.BlockSpec((4, 1000, H), lambda j: (0, j, 0)),
        out_shape=jax.ShapeDtypeStruct((4, N, H), jnp.float32),
    )(sir, sir, ai_partials, ai_partials, x, lnw2, lnb2)


def kernel(t, x, edge_index, W, b, ln_w, ln_b):
    del t
    wt = W.T
    perm = jnp.asarray(PERM, jnp.int32)
    sir, tbl = _matmul_relu(x, wt, b.reshape(1, H),
                            wt[:, perm], b[perm].reshape(1, H))

    n_pad = E_PAD - E
    rows = jnp.concatenate(
        [edge_index[0], jnp.full((n_pad,), GARBAGE_ROW, jnp.int32)])
    cols = jnp.concatenate(
        [edge_index[1] + jnp.int32(N), jnp.zeros((n_pad,), jnp.int32)])
    idx = jnp.stack([cols.reshape(E_PAD // K, K),
                     rows.reshape(E_PAD // K, K)], axis=1)

    ai_partials = _sc_scatter()(idx, tbl)

    out = _finalize(sir, ai_partials, x,
                    ln_w.reshape(1, H), ln_b.reshape(1, H))
    return out.reshape(4 * N, H)
